# baseline (speedup 1.0000x reference)
import functools
import jax
import jax.numpy as jnp
from jax import lax
from jax.experimental import pallas as pl
from jax.experimental.pallas import tpu as pltpu
from jax.experimental.pallas import tpu_sc as plsc

SEQ = 8192
D = 768
NPH = 9
B = 4
NC = 2
NS = 16
NW = NC * NS
ROWS_W = SEQ // NW
CH = 16
NJT = D // 128
NL = 128
ROW = B * D


def _sc_kernel_factory():
    mesh = plsc.VectorSubcoreMesh(core_axis_name="c", subcore_axis_name="s")

    @functools.partial(
        pl.kernel,
        mesh=mesh,
        out_type=jax.ShapeDtypeStruct((SEQ, ROW), jnp.float32),
        scratch_types=[
            pltpu.VMEM((CH, ROW), jnp.float32),
            pltpu.VMEM((NPH * B, NL), jnp.float32),
            pltpu.VMEM((NPH, D), jnp.float32),
        ],
    )
    def k(xv_hbm, qv_hbm, emb_hbm, out_hbm, xbuf, qbuf, embv):
        wid = lax.axis_index("s") * NC + lax.axis_index("c")
        pltpu.sync_copy(emb_hbm, embv)
        base = wid * ROWS_W

        def chunk_body(ci, carry):
            s0 = base + ci * CH
            j = s0 // NL
            l0 = s0 % NL
            for i in range(NPH):
                pltpu.sync_copy(
                    qv_hbm.at[pl.ds(i * (SEQ // NL * B) + j * B, B)],
                    qbuf.at[pl.ds(i * B, B)],
                )
            pltpu.sync_copy(xv_hbm.at[pl.ds(s0, CH)], xbuf)

            def row_body(t, inner):
                tv = jnp.full((16,), t, dtype=jnp.int32)
                for b in range(B):
                    ws = []
                    for i in range(NPH):
                        vec16 = qbuf[i * B + b, pl.ds(l0, 16)]
                        ws.append(vec16.at[tv].get(mode="promise_in_bounds"))
                    for jj in range(NJT):
                        for kk in range(NL // 16):
                            off = jj * (B * NL) + b * NL + kk * 16
                            acc = xbuf[t, pl.ds(off, 16)]
                            for i in range(NPH):
                                ev = embv[i, pl.ds(jj * NL + kk * 16, 16)]
                                acc = acc + ws[i] * ev
                            xbuf[t, pl.ds(off, 16)] = acc
                return inner

            lax.fori_loop(0, CH, row_body, 0)
            pltpu.sync_copy(xbuf, out_hbm.at[pl.ds(s0, CH)])
            return carry

        lax.fori_loop(0, ROWS_W // CH, chunk_body, 0)

    return k


def kernel(x, phase_one_hot, emb_table):
    seq, batch, d = x.shape
    xv2 = x.reshape(seq, batch, NJT, NL).transpose(0, 2, 1, 3).reshape(seq, ROW)
    qv2 = (phase_one_hot.reshape(seq // NL, NL, batch, NPH)
           .transpose(3, 0, 2, 1).reshape(NPH * (seq // NL) * batch, NL))
    out = _sc_kernel_factory()(xv2, qv2, emb_table)
    return (out.reshape(seq, NJT, batch, NL).transpose(0, 2, 1, 3)
            .reshape(seq, batch, d))


# speedup vs baseline: 35.6324x; 35.6324x over previous
import jax
import jax.numpy as jnp
from jax.experimental import pallas as pl
from jax.experimental.pallas import tpu as pltpu


def _body(x_ref, q_ref, emb_ref, out_ref):
    blk, batch, d = x_ref.shape
    n = q_ref.shape[0]
    pb = jnp.transpose(q_ref[...], (0, 2, 1)).reshape(n, blk * batch)
    s = jax.lax.dot_general(
        pb, emb_ref[...],
        dimension_numbers=(((0,), (0,)), ((), ())),
        preferred_element_type=jnp.float32,
    )
    out_ref[...] = x_ref[...] + s.reshape(blk, batch, d)


def kernel(x, phase_one_hot, emb_table):
    seq, batch, d = x.shape
    n = emb_table.shape[0]
    q = jnp.transpose(phase_one_hot, (2, 1, 0))
    blk = 1024
    grid = (seq // blk,)
    return pl.pallas_call(
        _body,
        grid=grid,
        in_specs=[
            pl.BlockSpec((blk, batch, d), lambda i: (i, 0, 0)),
            pl.BlockSpec((n, batch, blk), lambda i: (0, 0, i)),
            pl.BlockSpec((n, d), lambda i: (0, 0)),
        ],
        out_specs=pl.BlockSpec((blk, batch, d), lambda i: (i, 0, 0)),
        out_shape=jax.ShapeDtypeStruct((seq, batch, d), x.dtype),
        compiler_params=pltpu.CompilerParams(
            fuse_transposed_lhs_in_matmul=True,
        ),
    )(x, q, emb_table)
